# baseline (speedup 1.0000x reference)
import functools

import jax
import jax.numpy as jnp
import numpy as np
from jax import lax
from jax.experimental import pallas as pl
from jax.experimental.pallas import tpu as pltpu
from jax.experimental.pallas import tpu_sc as plsc

_VOCAB = 64
_SEQ = 512
_D = 128
_H = 768
_BATCH = 1024

_INFO = plsc.get_sparse_core_info()
_NC = _INFO.num_cores
_NS = _INFO.num_subcores
_NW = _NC * _NS
_TOK = _BATCH * _SEQ
_TPW = _TOK // _NW
_CH = 64
_NB = 4
_K = 2
_LANES = 16

_HALF = _SEQ // _NC
_NPHASE = 4
_QUART = _HALF // _NPHASE
_ROWS_W = _BATCH // _NS
_CPR = _QUART // _CH
_NCH_P = _ROWS_W * _CPR


def _sinusoid_pe_np():
    pos = np.arange(_SEQ)[:, None].astype(np.float32)
    i = np.arange(_D // 2)[None, :].astype(np.float32)
    ang = pos / np.power(10000.0, (2.0 * i) / float(_D))
    pe = np.zeros((_SEQ, _D), dtype=np.float32)
    pe[:, 0::2] = np.sin(ang)
    pe[:, 1::2] = np.cos(ang)
    return pe


_PE = _sinusoid_pe_np()

_L_BLK = 256


def _comb_body(tbl_ref, w_ref, b_ref, pe_ref, out_ref, red_ref):
    @pl.when(pl.program_id(0) == 0)
    def _():
        red_ref[...] = (
            jax.lax.dot_general(
                tbl_ref[...], w_ref[...], (((1,), (0,)), ((), ())),
                preferred_element_type=jnp.float32,
                precision=jax.lax.Precision.HIGHEST,
            )
            + b_ref[...][None, :]
        )
    out_ref[...] = red_ref[...][None, :, :] + pe_ref[...][:, None, :]


def _build_combined(matbert_table, W, b, pe):
    out3 = pl.pallas_call(
        _comb_body,
        grid=(_SEQ // _L_BLK,),
        in_specs=[
            pl.BlockSpec((_VOCAB, _H), lambda i: (0, 0)),
            pl.BlockSpec((_H, _D), lambda i: (0, 0)),
            pl.BlockSpec((_D,), lambda i: (0,)),
            pl.BlockSpec((_L_BLK, _D), lambda i: (i, 0)),
        ],
        out_specs=pl.BlockSpec((_L_BLK, _VOCAB, _D), lambda i: (i, 0, 0)),
        out_shape=jax.ShapeDtypeStruct((_SEQ, _VOCAB, _D), jnp.float32),
        scratch_shapes=[pltpu.VMEM((_VOCAB, _D), jnp.float32)],
    )(matbert_table, W, b, pe)
    return out3.reshape(_SEQ * _VOCAB, _D)


def _sc_gather(comb, x, posv):
    @functools.partial(
        pl.kernel,
        out_type=jax.ShapeDtypeStruct((_TOK, _D), jnp.float32),
        mesh=plsc.VectorSubcoreMesh(core_axis_name="c", subcore_axis_name="s"),
        scratch_types=(
            [pltpu.VMEM_SHARED((_QUART * _VOCAB, _D), jnp.float32) for _ in range(2)]
            + [pltpu.VMEM((_ROWS_W, 2 * _QUART), jnp.int32)]
            + [pltpu.VMEM((_QUART,), jnp.int32)]
            + [pltpu.VMEM((_CH,), jnp.int32) for _ in range(_NB)]
            + [pltpu.VMEM((_CH, _D), jnp.float32) for _ in range(_NB)]
            + [pltpu.SemaphoreType.DMA for _ in range(2 * _NB + 2)]
        ),
    )
    def run(comb_hbm, x_hbm, pos_hbm, out_hbm, comb_sh0, comb_sh1, x_all, pos_v, *bufs):
        comb_shs = (comb_sh0, comb_sh1)
        idxb = bufs[:_NB]
        rows = bufs[_NB : 2 * _NB]
        sg = bufs[2 * _NB : 3 * _NB]
        ss = bufs[3 * _NB : 4 * _NB]
        stg = bufs[4 * _NB :]
        ci = lax.axis_index("c")
        si = lax.axis_index("s")

        pltpu.sync_copy(pos_hbm, pos_v)

        def stage(p):
            return pltpu.make_async_copy(
                comb_hbm.at[
                    pl.ds(
                        (ci * _NPHASE + p) * (_QUART * _VOCAB),
                        _QUART * _VOCAB,
                    )
                ],
                comb_shs[p % 2],
                stg[p % 2],
            )

        @pl.when(si == 0)
        def _():
            stage(0).start()

        for p in range(_NPHASE):
            comb_sh = comb_shs[p % 2]

            if p % 2 == 0:
                pltpu.sync_copy(
                    x_hbm.at[
                        pl.ds(si * _ROWS_W, _ROWS_W),
                        pl.ds(ci * _HALF + p * _QUART, 2 * _QUART),
                    ],
                    x_all,
                )

            @pl.when(si == 0)
            def _():
                stage(p).wait()

            plsc.subcore_barrier()

            @pl.when(si == 0)
            def _():
                if p + 1 < _NPHASE:
                    stage(p + 1).start()

            def fill_idx(b, row, win):
                dst = idxb[b]
                for j in range(_CH // _LANES):
                    o = win * _CH + j * _LANES
                    dst[pl.ds(j * _LANES, _LANES)] = (
                        x_all[row, pl.ds((p % 2) * _QUART + o, _LANES)]
                        + pos_v[pl.ds(o, _LANES)]
                    )

            def gdesc(b):
                return pltpu.make_async_copy(
                    comb_sh.at[idxb[b]], rows[b], sg[b]
                )

            def sdesc(b, row, win):
                base = (
                    (si * _ROWS_W + row) * _SEQ
                    + ci * _HALF
                    + p * _QUART
                    + win * _CH
                )
                return pltpu.make_async_copy(
                    rows[b], out_hbm.at[pl.ds(base, _CH)], ss[b]
                )

            for k0 in range(_K):
                if p > 0:
                    sdesc(k0 % _NB, 0, 0).wait()
                fill_idx(k0 % _NB, k0 // _CPR, k0 % _CPR)
                gdesc(k0 % _NB).start()

            def outer(i, carry):
                for b in range(_NB):
                    k = i * _NB + b
                    row = i * (_NB // _CPR) + b // _CPR
                    win = b % _CPR
                    pf = k + _K
                    bp = (b + _K) % _NB
                    pfrow = i * (_NB // _CPR) + (b + _K) // _CPR
                    pfwin = (b + _K) % _CPR

                    @pl.when(pf < _NCH_P)
                    def _():
                        if p == 0:
                            @pl.when(pf >= _NB)
                            def _():
                                sdesc(bp, 0, 0).wait()
                        else:
                            sdesc(bp, 0, 0).wait()

                        fill_idx(bp, pfrow, pfwin)
                        gdesc(bp).start()

                    gdesc(b).wait()
                    sdesc(b, row, win).start()
                return carry

            lax.fori_loop(0, _NCH_P // _NB, outer, 0)

        for b in range(_NB):
            pltpu.make_async_copy(
                rows[b], out_hbm.at[pl.ds(b * _CH, _CH)], ss[b]
            ).wait()

    return run(comb, x, posv)


def kernel(x, matbert_table, W, b):
    pe = jnp.asarray(_PE)
    posv = jnp.arange(_QUART, dtype=jnp.int32) * _VOCAB
    comb = _build_combined(matbert_table, W, b, pe)
    out = _sc_gather(comb, x, posv)
    return out.reshape(_BATCH, _SEQ, _D)


# speedup vs baseline: 1.0082x; 1.0082x over previous
import functools

import jax
import jax.numpy as jnp
import numpy as np
from jax import lax
from jax.experimental import pallas as pl
from jax.experimental.pallas import tpu as pltpu
from jax.experimental.pallas import tpu_sc as plsc

_VOCAB = 64
_SEQ = 512
_D = 128
_H = 768
_BATCH = 1024

_INFO = plsc.get_sparse_core_info()
_NC = _INFO.num_cores
_NS = _INFO.num_subcores
_NW = _NC * _NS
_TOK = _BATCH * _SEQ
_TPW = _TOK // _NW
_CH = 64
_NB = 4
_K = 2
_LANES = 16

_HALF = _SEQ // _NC
_NPHASE = 4
_QUART = _HALF // _NPHASE
_ROWS_W = _BATCH // _NS
_CPR = _QUART // _CH
_NCH_P = _ROWS_W * _CPR


def _sinusoid_pe_np():
    pos = np.arange(_SEQ)[:, None].astype(np.float32)
    i = np.arange(_D // 2)[None, :].astype(np.float32)
    ang = pos / np.power(10000.0, (2.0 * i) / float(_D))
    pe = np.zeros((_SEQ, _D), dtype=np.float32)
    pe[:, 0::2] = np.sin(ang)
    pe[:, 1::2] = np.cos(ang)
    return pe


_PE = _sinusoid_pe_np()

_L_BLK = 128


def _comb_body(tbl_ref, w_ref, b_ref, pe_ref, out_ref, red_ref):
    @pl.when(pl.program_id(0) == 0)
    def _():
        red_ref[...] = (
            jax.lax.dot_general(
                tbl_ref[...], w_ref[...], (((1,), (0,)), ((), ())),
                preferred_element_type=jnp.float32,
                precision=jax.lax.Precision.HIGHEST,
            )
            + b_ref[...][None, :]
        )
    out_ref[...] = red_ref[...][None, :, :] + pe_ref[...][:, None, :]


def _build_combined(matbert_table, W, b, pe):
    out3 = pl.pallas_call(
        _comb_body,
        grid=(_SEQ // _L_BLK,),
        in_specs=[
            pl.BlockSpec((_VOCAB, _H), lambda i: (0, 0)),
            pl.BlockSpec((_H, _D), lambda i: (0, 0)),
            pl.BlockSpec((_D,), lambda i: (0,)),
            pl.BlockSpec((_L_BLK, _D), lambda i: (i, 0)),
        ],
        out_specs=pl.BlockSpec((_L_BLK, _VOCAB, _D), lambda i: (i, 0, 0)),
        out_shape=jax.ShapeDtypeStruct((_SEQ, _VOCAB, _D), jnp.float32),
        scratch_shapes=[pltpu.VMEM((_VOCAB, _D), jnp.float32)],
    )(matbert_table, W, b, pe)
    return out3.reshape(_SEQ * _VOCAB, _D)


def _sc_gather(comb, x, posv):
    @functools.partial(
        pl.kernel,
        out_type=jax.ShapeDtypeStruct((_TOK, _D), jnp.float32),
        mesh=plsc.VectorSubcoreMesh(core_axis_name="c", subcore_axis_name="s"),
        scratch_types=(
            [pltpu.VMEM_SHARED((_QUART * _VOCAB, _D), jnp.float32) for _ in range(2)]
            + [pltpu.VMEM((_ROWS_W, 2 * _QUART), jnp.int32)]
            + [pltpu.VMEM((_QUART,), jnp.int32)]
            + [pltpu.VMEM((_CH,), jnp.int32) for _ in range(_NB)]
            + [pltpu.VMEM((_CH, _D), jnp.float32) for _ in range(_NB)]
            + [pltpu.SemaphoreType.DMA for _ in range(2 * _NB + 2)]
        ),
    )
    def run(comb_hbm, x_hbm, pos_hbm, out_hbm, comb_sh0, comb_sh1, x_all, pos_v, *bufs):
        comb_shs = (comb_sh0, comb_sh1)
        idxb = bufs[:_NB]
        rows = bufs[_NB : 2 * _NB]
        sg = bufs[2 * _NB : 3 * _NB]
        ss = bufs[3 * _NB : 4 * _NB]
        stg = bufs[4 * _NB :]
        ci = lax.axis_index("c")
        si = lax.axis_index("s")

        pltpu.sync_copy(pos_hbm, pos_v)

        _SEG = _QUART * _VOCAB // _NS

        def stage(p):
            return pltpu.make_async_copy(
                comb_hbm.at[
                    pl.ds(
                        (ci * _NPHASE + p) * (_QUART * _VOCAB) + si * _SEG,
                        _SEG,
                    )
                ],
                comb_shs[p % 2].at[pl.ds(si * _SEG, _SEG)],
                stg[p % 2],
            )

        stage(0).start()

        for p in range(_NPHASE):
            comb_sh = comb_shs[p % 2]

            if p % 2 == 0:
                pltpu.sync_copy(
                    x_hbm.at[
                        pl.ds(si * _ROWS_W, _ROWS_W),
                        pl.ds(ci * _HALF + p * _QUART, 2 * _QUART),
                    ],
                    x_all,
                )

            stage(p).wait()

            plsc.subcore_barrier()

            if p + 1 < _NPHASE:
                stage(p + 1).start()

            def fill_idx(b, row, win):
                dst = idxb[b]
                for j in range(_CH // _LANES):
                    o = win * _CH + j * _LANES
                    dst[pl.ds(j * _LANES, _LANES)] = (
                        x_all[row, pl.ds((p % 2) * _QUART + o, _LANES)]
                        + pos_v[pl.ds(o, _LANES)]
                    )

            def gdesc(b):
                return pltpu.make_async_copy(
                    comb_sh.at[idxb[b]], rows[b], sg[b]
                )

            def sdesc(b, row, win):
                base = (
                    (si * _ROWS_W + row) * _SEQ
                    + ci * _HALF
                    + p * _QUART
                    + win * _CH
                )
                return pltpu.make_async_copy(
                    rows[b], out_hbm.at[pl.ds(base, _CH)], ss[b]
                )

            for k0 in range(_K):
                if p > 0:
                    sdesc(k0 % _NB, 0, 0).wait()
                fill_idx(k0 % _NB, k0 // _CPR, k0 % _CPR)
                gdesc(k0 % _NB).start()

            def outer(i, carry):
                for b in range(_NB):
                    k = i * _NB + b
                    row = i * (_NB // _CPR) + b // _CPR
                    win = b % _CPR
                    pf = k + _K
                    bp = (b + _K) % _NB
                    pfrow = i * (_NB // _CPR) + (b + _K) // _CPR
                    pfwin = (b + _K) % _CPR

                    @pl.when(pf < _NCH_P)
                    def _():
                        if p == 0:
                            @pl.when(pf >= _NB)
                            def _():
                                sdesc(bp, 0, 0).wait()
                        else:
                            sdesc(bp, 0, 0).wait()

                        fill_idx(bp, pfrow, pfwin)
                        gdesc(bp).start()

                    gdesc(b).wait()
                    sdesc(b, row, win).start()
                return carry

            lax.fori_loop(0, _NCH_P // _NB, outer, 0)

        for b in range(_NB):
            pltpu.make_async_copy(
                rows[b], out_hbm.at[pl.ds(b * _CH, _CH)], ss[b]
            ).wait()

    return run(comb, x, posv)


def kernel(x, matbert_table, W, b):
    pe = jnp.asarray(_PE)
    posv = jnp.arange(_QUART, dtype=jnp.int32) * _VOCAB
    comb = _build_combined(matbert_table, W, b, pe)
    out = _sc_gather(comb, x, posv)
    return out.reshape(_BATCH, _SEQ, _D)


# speedup vs baseline: 1.0170x; 1.0087x over previous
import functools

import jax
import jax.numpy as jnp
import numpy as np
from jax import lax
from jax.experimental import pallas as pl
from jax.experimental.pallas import tpu as pltpu
from jax.experimental.pallas import tpu_sc as plsc

_VOCAB = 64
_SEQ = 512
_D = 128
_H = 768
_BATCH = 1024

_INFO = plsc.get_sparse_core_info()
_NC = _INFO.num_cores
_NS = _INFO.num_subcores
_NW = _NC * _NS
_TOK = _BATCH * _SEQ
_TPW = _TOK // _NW
_CH = 64
_NB = 4
_K = 2
_LANES = 16

_HALF = _SEQ // _NC
_NPHASE = 4
_QUART = _HALF // _NPHASE
_ROWS_W = _BATCH // _NS
_CPR = _QUART // _CH
_NCH_P = _ROWS_W * _CPR


def _sinusoid_pe_np():
    pos = np.arange(_SEQ)[:, None].astype(np.float32)
    i = np.arange(_D // 2)[None, :].astype(np.float32)
    ang = pos / np.power(10000.0, (2.0 * i) / float(_D))
    pe = np.zeros((_SEQ, _D), dtype=np.float32)
    pe[:, 0::2] = np.sin(ang)
    pe[:, 1::2] = np.cos(ang)
    return pe


_PE = _sinusoid_pe_np()

_L_BLK = 128


def _comb_body(tbl_ref, w_ref, b_ref, pe_ref, out_ref, red_ref):
    @pl.when(pl.program_id(0) == 0)
    def _():
        red_ref[...] = (
            jax.lax.dot_general(
                tbl_ref[...], w_ref[...], (((1,), (0,)), ((), ())),
                preferred_element_type=jnp.float32,
                precision=jax.lax.Precision.HIGHEST,
            )
            + b_ref[...][None, :]
        )
    out_ref[...] = red_ref[...][None, :, :] + pe_ref[...][:, None, :]


def _build_combined(matbert_table, W, b, pe):
    out3 = pl.pallas_call(
        _comb_body,
        grid=(_SEQ // _L_BLK,),
        in_specs=[
            pl.BlockSpec((_VOCAB, _H), lambda i: (0, 0)),
            pl.BlockSpec((_H, _D), lambda i: (0, 0)),
            pl.BlockSpec((_D,), lambda i: (0,)),
            pl.BlockSpec((_L_BLK, _D), lambda i: (i, 0)),
        ],
        out_specs=pl.BlockSpec((_L_BLK, _VOCAB, _D), lambda i: (i, 0, 0)),
        out_shape=jax.ShapeDtypeStruct((_SEQ, _VOCAB, _D), jnp.float32),
        scratch_shapes=[pltpu.VMEM((_VOCAB, _D), jnp.float32)],
    )(matbert_table, W, b, pe)
    return out3.reshape(_SEQ * _VOCAB, _D)


def _sc_gather(comb, x, posv):
    @functools.partial(
        pl.kernel,
        out_type=jax.ShapeDtypeStruct((_TOK, _D), jnp.float32),
        mesh=plsc.VectorSubcoreMesh(core_axis_name="c", subcore_axis_name="s"),
        scratch_types=(
            [pltpu.VMEM_SHARED((_QUART * _VOCAB, _D), jnp.float32) for _ in range(2)]
            + [pltpu.VMEM((_ROWS_W, 2 * _QUART), jnp.int32) for _ in range(2)]
            + [pltpu.VMEM((_QUART,), jnp.int32)]
            + [pltpu.VMEM((_CH,), jnp.int32) for _ in range(_NB)]
            + [pltpu.VMEM((_CH, _D), jnp.float32) for _ in range(_NB)]
            + [pltpu.SemaphoreType.DMA for _ in range(2 * _NB + 3)]
        ),
    )
    def run(comb_hbm, x_hbm, pos_hbm, out_hbm, comb_sh0, comb_sh1,
            x_all0, x_all1, pos_v, *bufs):
        comb_shs = (comb_sh0, comb_sh1)
        x_alls = (x_all0, x_all1)
        idxb = bufs[:_NB]
        rows = bufs[_NB : 2 * _NB]
        sg = bufs[2 * _NB : 3 * _NB]
        ss = bufs[3 * _NB : 4 * _NB]
        stg = bufs[4 * _NB : 4 * _NB + 2]
        sx = bufs[4 * _NB + 2]
        ci = lax.axis_index("c")
        si = lax.axis_index("s")

        pltpu.sync_copy(pos_hbm, pos_v)

        _SEG = _QUART * _VOCAB // _NS

        def stage(p):
            return pltpu.make_async_copy(
                comb_hbm.at[
                    pl.ds(
                        (ci * _NPHASE + p) * (_QUART * _VOCAB) + si * _SEG,
                        _SEG,
                    )
                ],
                comb_shs[p % 2].at[pl.ds(si * _SEG, _SEG)],
                stg[p % 2],
            )

        stage(0).start()

        def xdesc(h):
            return pltpu.make_async_copy(
                x_hbm.at[
                    pl.ds(si * _ROWS_W, _ROWS_W),
                    pl.ds(ci * _HALF + h * 2 * _QUART, 2 * _QUART),
                ],
                x_alls[h],
                sx,
            )

        xdesc(0).start()
        xdesc(1).start()

        for p in range(_NPHASE):
            comb_sh = comb_shs[p % 2]
            x_all = x_alls[p // 2]

            if p == 0:
                xdesc(0).wait()
                xdesc(1).wait()

            stage(p).wait()

            plsc.subcore_barrier()

            if p + 1 < _NPHASE:
                stage(p + 1).start()

            def fill_idx(b, row, win):
                dst = idxb[b]
                for j in range(_CH // _LANES):
                    o = win * _CH + j * _LANES
                    dst[pl.ds(j * _LANES, _LANES)] = (
                        x_all[row, pl.ds((p % 2) * _QUART + o, _LANES)]
                        + pos_v[pl.ds(o, _LANES)]
                    )

            def gdesc(b):
                return pltpu.make_async_copy(
                    comb_sh.at[idxb[b]], rows[b], sg[b]
                )

            def sdesc(b, row, win):
                base = (
                    (si * _ROWS_W + row) * _SEQ
                    + ci * _HALF
                    + p * _QUART
                    + win * _CH
                )
                return pltpu.make_async_copy(
                    rows[b], out_hbm.at[pl.ds(base, _CH)], ss[b]
                )

            for k0 in range(_K):
                if p > 0:
                    sdesc(k0 % _NB, 0, 0).wait()
                fill_idx(k0 % _NB, k0 // _CPR, k0 % _CPR)
                gdesc(k0 % _NB).start()

            def outer(i, carry):
                for b in range(_NB):
                    k = i * _NB + b
                    row = i * (_NB // _CPR) + b // _CPR
                    win = b % _CPR
                    pf = k + _K
                    bp = (b + _K) % _NB
                    pfrow = i * (_NB // _CPR) + (b + _K) // _CPR
                    pfwin = (b + _K) % _CPR

                    @pl.when(pf < _NCH_P)
                    def _():
                        if p == 0:
                            @pl.when(pf >= _NB)
                            def _():
                                sdesc(bp, 0, 0).wait()
                        else:
                            sdesc(bp, 0, 0).wait()

                        fill_idx(bp, pfrow, pfwin)
                        gdesc(bp).start()

                    gdesc(b).wait()
                    sdesc(b, row, win).start()
                return carry

            lax.fori_loop(0, _NCH_P // _NB, outer, 0)

        for b in range(_NB):
            pltpu.make_async_copy(
                rows[b], out_hbm.at[pl.ds(b * _CH, _CH)], ss[b]
            ).wait()

    return run(comb, x, posv)


def kernel(x, matbert_table, W, b):
    pe = jnp.asarray(_PE)
    posv = jnp.arange(_QUART, dtype=jnp.int32) * _VOCAB
    comb = _build_combined(matbert_table, W, b, pe)
    out = _sc_gather(comb, x, posv)
    return out.reshape(_BATCH, _SEQ, _D)


# speedup vs baseline: 1.0246x; 1.0075x over previous
import functools

import jax
import jax.numpy as jnp
import numpy as np
from jax import lax
from jax.experimental import pallas as pl
from jax.experimental.pallas import tpu as pltpu
from jax.experimental.pallas import tpu_sc as plsc

_VOCAB = 64
_SEQ = 512
_D = 128
_H = 768
_BATCH = 1024

_INFO = plsc.get_sparse_core_info()
_NC = _INFO.num_cores
_NS = _INFO.num_subcores
_NW = _NC * _NS
_TOK = _BATCH * _SEQ
_TPW = _TOK // _NW
_CH = 64
_NB = 4
_K = 2
_LANES = 16

_HALF = _SEQ // _NC
_NPHASE = 4
_QUART = _HALF // _NPHASE
_ROWS_W = _BATCH // _NS
_CPR = _QUART // _CH
_NCH_P = _ROWS_W * _CPR


def _sinusoid_pe_np():
    pos = np.arange(_SEQ)[:, None].astype(np.float32)
    i = np.arange(_D // 2)[None, :].astype(np.float32)
    ang = pos / np.power(10000.0, (2.0 * i) / float(_D))
    pe = np.zeros((_SEQ, _D), dtype=np.float32)
    pe[:, 0::2] = np.sin(ang)
    pe[:, 1::2] = np.cos(ang)
    return pe


_PE = _sinusoid_pe_np()

_L_BLK = 128


def _comb_body(tbl_ref, w_ref, b_ref, pe_ref, out_ref, red_ref):
    @pl.when(pl.program_id(0) == 0)
    def _():
        red_ref[...] = (
            jax.lax.dot_general(
                tbl_ref[...], w_ref[...], (((1,), (0,)), ((), ())),
                preferred_element_type=jnp.float32,
                precision=jax.lax.Precision.HIGHEST,
            )
            + b_ref[...][None, :]
        )
    out_ref[...] = red_ref[...][None, :, :] + pe_ref[...][:, None, :]


def _build_combined(matbert_table, W, b, pe):
    out3 = pl.pallas_call(
        _comb_body,
        grid=(_SEQ // _L_BLK,),
        in_specs=[
            pl.BlockSpec((_VOCAB, _H), lambda i: (0, 0)),
            pl.BlockSpec((_H, _D), lambda i: (0, 0)),
            pl.BlockSpec((_D,), lambda i: (0,)),
            pl.BlockSpec((_L_BLK, _D), lambda i: (i, 0)),
        ],
        out_specs=pl.BlockSpec((_L_BLK, _VOCAB, _D), lambda i: (i, 0, 0)),
        out_shape=jax.ShapeDtypeStruct((_SEQ, _VOCAB, _D), jnp.float32),
        scratch_shapes=[pltpu.VMEM((_VOCAB, _D), jnp.float32)],
    )(matbert_table, W, b, pe)
    return out3.reshape(_SEQ * _VOCAB, _D)


def _sc_gather(comb, x, posv):
    @functools.partial(
        pl.kernel,
        out_type=jax.ShapeDtypeStruct((_TOK, _D), jnp.float32),
        mesh=plsc.VectorSubcoreMesh(core_axis_name="c", subcore_axis_name="s"),
        scratch_types=(
            [pltpu.VMEM_SHARED((_QUART * _VOCAB, _D), jnp.float32) for _ in range(2)]
            + [pltpu.VMEM((_ROWS_W, 2 * _QUART), jnp.int32) for _ in range(2)]
            + [pltpu.VMEM((_QUART,), jnp.int32)]
            + [pltpu.VMEM((_CH,), jnp.int32) for _ in range(_NB)]
            + [pltpu.VMEM((_CH, _D), jnp.float32) for _ in range(_NB)]
            + [pltpu.SemaphoreType.DMA for _ in range(2 * _NB + 3)]
        ),
    )
    def run(comb_hbm, x_hbm, pos_hbm, out_hbm, comb_sh0, comb_sh1,
            x_all0, x_all1, pos_v, *bufs):
        comb_shs = (comb_sh0, comb_sh1)
        x_alls = (x_all0, x_all1)
        idxb = bufs[:_NB]
        rows = bufs[_NB : 2 * _NB]
        sg = bufs[2 * _NB : 3 * _NB]
        ss = bufs[3 * _NB : 4 * _NB]
        stg = bufs[4 * _NB : 4 * _NB + 2]
        sx = bufs[4 * _NB + 2]
        ci = lax.axis_index("c")
        si = lax.axis_index("s")

        _SEG = _QUART * _VOCAB // _NS

        def stage(p):
            return pltpu.make_async_copy(
                comb_hbm.at[
                    pl.ds(
                        (ci * _NPHASE + p) * (_QUART * _VOCAB) + si * _SEG,
                        _SEG,
                    )
                ],
                comb_shs[p % 2].at[pl.ds(si * _SEG, _SEG)],
                stg[p % 2],
            )

        stage(0).start()

        def xdesc(h):
            return pltpu.make_async_copy(
                x_hbm.at[
                    pl.ds(si * _ROWS_W, _ROWS_W),
                    pl.ds(ci * _HALF + h * 2 * _QUART, 2 * _QUART),
                ],
                x_alls[h],
                sx,
            )

        xdesc(0).start()
        xdesc(1).start()
        pltpu.sync_copy(pos_hbm, pos_v)

        for p in range(_NPHASE):
            comb_sh = comb_shs[p % 2]
            x_all = x_alls[p // 2]

            if p == 0:
                xdesc(0).wait()
                xdesc(1).wait()

            stage(p).wait()

            plsc.subcore_barrier()

            if p + 1 < _NPHASE:
                stage(p + 1).start()

            def fill_idx(b, row, win):
                dst = idxb[b]
                for j in range(_CH // _LANES):
                    o = win * _CH + j * _LANES
                    dst[pl.ds(j * _LANES, _LANES)] = (
                        x_all[row, pl.ds((p % 2) * _QUART + o, _LANES)]
                        + pos_v[pl.ds(o, _LANES)]
                    )

            def gdesc(b):
                return pltpu.make_async_copy(
                    comb_sh.at[idxb[b]], rows[b], sg[b]
                )

            def sdesc(b, row, win):
                base = (
                    (si * _ROWS_W + row) * _SEQ
                    + ci * _HALF
                    + p * _QUART
                    + win * _CH
                )
                return pltpu.make_async_copy(
                    rows[b], out_hbm.at[pl.ds(base, _CH)], ss[b]
                )

            for k0 in range(_K):
                if p > 0:
                    sdesc(k0 % _NB, 0, 0).wait()
                fill_idx(k0 % _NB, k0 // _CPR, k0 % _CPR)
                gdesc(k0 % _NB).start()

            def outer(i, carry):
                for b in range(_NB):
                    k = i * _NB + b
                    row = i * (_NB // _CPR) + b // _CPR
                    win = b % _CPR
                    pf = k + _K
                    bp = (b + _K) % _NB
                    pfrow = i * (_NB // _CPR) + (b + _K) // _CPR
                    pfwin = (b + _K) % _CPR

                    @pl.when(pf < _NCH_P)
                    def _():
                        if p == 0:
                            @pl.when(pf >= _NB)
                            def _():
                                sdesc(bp, 0, 0).wait()
                        else:
                            sdesc(bp, 0, 0).wait()

                        fill_idx(bp, pfrow, pfwin)
                        gdesc(bp).start()

                    gdesc(b).wait()
                    sdesc(b, row, win).start()
                return carry

            lax.fori_loop(0, _NCH_P // _NB, outer, 0)

        for b in range(_NB):
            pltpu.make_async_copy(
                rows[b], out_hbm.at[pl.ds(b * _CH, _CH)], ss[b]
            ).wait()

    return run(comb, x, posv)


def kernel(x, matbert_table, W, b):
    pe = jnp.asarray(_PE)
    posv = jnp.arange(_QUART, dtype=jnp.int32) * _VOCAB
    comb = _build_combined(matbert_table, W, b, pe)
    out = _sc_gather(comb, x, posv)
    return out.reshape(_BATCH, _SEQ, _D)
